# baseline (speedup 1.0000x reference)
import functools

import jax
import jax.numpy as jnp
from jax import lax
from jax.experimental import pallas as pl
from jax.experimental.pallas import tpu as pltpu
from jax.experimental.pallas import tpu_sc as plsc

B = 16
N = 16384
C = 64
S = 1024
L = 16
CHUNKS = N // L
UNROLL = 8

BIG_I32 = 2**30


def _xgather(xflat_hbm, xidx, xg, sem):
    cps = [pltpu.async_copy(xflat_hbm.at[xidx.at[q]],
                            xg.at[pl.ds(q * 128, 128)], sem)
           for q in range(8)]
    for cp in cps:
        cp.wait()


def _fps_body(p3_hbm, xflat_hbm, p3_out, xs_out,
              px, py, pz, dist, idxb, psx, psy, psz, xidx, xg, sem):
    c = lax.axis_index("c")
    s = lax.axis_index("s")
    b = c * 8 + s

    @pl.when(s < 8)
    def _():
        pltpu.sync_copy(p3_hbm.at[pl.ds((b * 3 + 0) * N, N)], px)
        pltpu.sync_copy(p3_hbm.at[pl.ds((b * 3 + 1) * N, N)], py)
        pltpu.sync_copy(p3_hbm.at[pl.ds((b * 3 + 2) * N, N)], pz)

        inf16 = jnp.full((L,), jnp.inf, dtype=jnp.float32)

        def init_body(j, _):
            base = j * (L * UNROLL)
            for u in range(UNROLL):
                dist[pl.ds(base + u * L, L)] = inf16
            return 0

        lax.fori_loop(0, CHUNKS // UNROLL, init_body, 0)

        lane = lax.iota(jnp.int32, L)
        lane0 = lane == 0

        def fps_iter(si, f):
            fv = jnp.full((L,), f, dtype=jnp.int32)
            plsc.store_scatter(idxb, [jnp.full((L,), si, dtype=jnp.int32)],
                               fv, mask=lane0)
            cx = plsc.load_gather(px, [fv])
            cy = plsc.load_gather(py, [fv])
            cz = plsc.load_gather(pz, [fv])

            def chunk_body(j, carry):
                vmaxs, jbests = carry
                base = j * (L * UNROLL)
                jv = jnp.full((L,), j, dtype=jnp.int32)
                vmaxs = list(vmaxs)
                jbests = list(jbests)
                for u in range(UNROLL):
                    o = base + u * L
                    dx = px[pl.ds(o, L)] - cx
                    dy = py[pl.ds(o, L)] - cy
                    dz = pz[pl.ds(o, L)] - cz
                    d = dx * dx + dy * dy + dz * dz
                    nd = jnp.minimum(dist[pl.ds(o, L)], d)
                    dist[pl.ds(o, L)] = nd
                    gt = nd > vmaxs[u]
                    vmaxs[u] = jnp.where(gt, nd, vmaxs[u])
                    jbests[u] = jnp.where(gt, jv, jbests[u])
                return tuple(vmaxs), tuple(jbests)

            neg1 = jnp.full((L,), -1.0, dtype=jnp.float32)
            zero = jnp.zeros((L,), dtype=jnp.int32)
            vmaxs, jbests = lax.fori_loop(
                0, CHUNKS // UNROLL, chunk_body,
                ((neg1,) * UNROLL, (zero,) * UNROLL))
            pairs = [(vmaxs[u], jbests[u] * (L * UNROLL) + (u * L) + lane)
                     for u in range(UNROLL)]
            while len(pairs) > 1:
                nxt = []
                for a in range(0, len(pairs), 2):
                    (va, ia), (vb, ib) = pairs[a], pairs[a + 1]
                    bwin = (vb > va) | ((vb == va) & (ib < ia))
                    nxt.append((jnp.where(bwin, vb, va),
                                jnp.where(bwin, ib, ia)))
                pairs = nxt
            vmax, vidx = pairs[0]
            m = jnp.max(vmax)
            cand = jnp.where(vmax == m, vidx, BIG_I32)
            return jnp.min(cand)

        lax.fori_loop(0, S, fps_iter, jnp.int32(0))

        def ps_body(j, _):
            base = j * L
            iv = idxb[pl.ds(base, L)]
            psx[pl.ds(base, L)] = plsc.load_gather(px, [iv])
            psy[pl.ds(base, L)] = plsc.load_gather(py, [iv])
            psz[pl.ds(base, L)] = plsc.load_gather(pz, [iv])
            return 0

        lax.fori_loop(0, S // L, ps_body, 0)
        pltpu.sync_copy(psx, p3_out.at[pl.ds((b * 3 + 0) * S, S)])
        pltpu.sync_copy(psy, p3_out.at[pl.ds((b * 3 + 1) * S, S)])
        pltpu.sync_copy(psz, p3_out.at[pl.ds((b * 3 + 2) * S, S)])

        def chan_body(ci, _):
            off = (b * C + ci) * N

            def bld(k, _):
                q = k // 8
                r = k - q * 8
                xidx[q, pl.ds(r * L, L)] = idxb[pl.ds(k * L, L)] + off
                return 0

            lax.fori_loop(0, S // L, bld, 0)

            _xgather(xflat_hbm, xidx, xg, sem)
            pltpu.sync_copy(xg, xs_out.at[pl.ds((b * C + ci) * S, S)])
            return 0

        lax.fori_loop(0, C, chan_body, 0)


@jax.jit
def kernel(p, x):
    p3 = jnp.transpose(p, (0, 2, 1)).reshape(B * 3 * N)
    xflat = x.reshape(B * C * N)

    mesh = plsc.VectorSubcoreMesh(core_axis_name="c", subcore_axis_name="s",
                                  num_cores=2, num_subcores=16)
    fps = pl.kernel(
        _fps_body,
        out_type=(
            jax.ShapeDtypeStruct((B * 3 * S,), jnp.float32),
            jax.ShapeDtypeStruct((B * C * S,), jnp.float32),
        ),
        mesh=mesh,
        compiler_params=pltpu.CompilerParams(needs_layout_passes=False),
        scratch_types=[
            pltpu.VMEM((N,), jnp.float32),
            pltpu.VMEM((N,), jnp.float32),
            pltpu.VMEM((N,), jnp.float32),
            pltpu.VMEM((N,), jnp.float32),
            pltpu.VMEM((S,), jnp.int32),
            pltpu.VMEM((S,), jnp.float32),
            pltpu.VMEM((S,), jnp.float32),
            pltpu.VMEM((S,), jnp.float32),
            pltpu.VMEM((8, 128), jnp.int32),
            pltpu.VMEM((S,), jnp.float32),
            pltpu.SemaphoreType.DMA,
        ],
    )
    p3_s, x_s = fps(p3, xflat)
    p_s = jnp.transpose(p3_s.reshape(B, 3, S), (0, 2, 1))
    return (p_s, x_s.reshape(B, C, S))


# speedup vs baseline: 3.0125x; 3.0125x over previous
import functools

import jax
import jax.numpy as jnp
from jax import lax
from jax.experimental import pallas as pl
from jax.experimental.pallas import tpu as pltpu
from jax.experimental.pallas import tpu_sc as plsc

B = 16
N = 16384
C = 64
S = 1024
L = 16
CHUNKS = N // L
UNROLL = 8

BIG_I32 = 2**30


def _xgather(xflat_hbm, xidx, xg, sem):
    cps = [pltpu.async_copy(xflat_hbm.at[xidx.at[q]],
                            xg.at[pl.ds(q * 128, 128)], sem)
           for q in range(8)]
    for cp in cps:
        cp.wait()


def _fps_body(p3_hbm, xflat_hbm, p3_out, xs_out,
              px, py, pz, dist, idxb, psx, psy, psz, xidx, xg, sem):
    c = lax.axis_index("c")
    s = lax.axis_index("s")
    b = c * 8 + s

    @pl.when(s < 8)
    def _():
        pltpu.sync_copy(p3_hbm.at[pl.ds((b * 3 + 0) * N, N)], px)
        pltpu.sync_copy(p3_hbm.at[pl.ds((b * 3 + 1) * N, N)], py)
        pltpu.sync_copy(p3_hbm.at[pl.ds((b * 3 + 2) * N, N)], pz)

        inf16 = jnp.full((L,), jnp.inf, dtype=jnp.float32)

        def init_body(j, _):
            base = j * (L * UNROLL)
            for u in range(UNROLL):
                dist[pl.ds(base + u * L, L)] = inf16
            return 0

        lax.fori_loop(0, CHUNKS // UNROLL, init_body, 0)

        lane = lax.iota(jnp.int32, L)
        lane0 = lane == 0

        def fps_iter(si, f):
            fv = jnp.full((L,), f, dtype=jnp.int32)
            plsc.store_scatter(idxb, [jnp.full((L,), si, dtype=jnp.int32)],
                               fv, mask=lane0)
            cx = plsc.load_gather(px, [fv])
            cy = plsc.load_gather(py, [fv])
            cz = plsc.load_gather(pz, [fv])

            def chunk_body(j, carry):
                vmaxs, jbests = carry
                base = j * (L * UNROLL)
                jv = jnp.full((L,), j, dtype=jnp.int32)
                vmaxs = list(vmaxs)
                jbests = list(jbests)
                R = range(UNROLL)
                o = [base + u * L for u in R]
                xs = [px[pl.ds(o[u], L)] for u in R]
                ys = [py[pl.ds(o[u], L)] for u in R]
                zs = [pz[pl.ds(o[u], L)] for u in R]
                dv = [dist[pl.ds(o[u], L)] for u in R]
                dx = [xs[u] - cx for u in R]
                dy = [ys[u] - cy for u in R]
                dz = [zs[u] - cz for u in R]
                d = [dx[u] * dx[u] for u in R]
                d = [d[u] + dy[u] * dy[u] for u in R]
                d = [d[u] + dz[u] * dz[u] for u in R]
                nd = [jnp.minimum(dv[u], d[u]) for u in R]
                for u in R:
                    dist[pl.ds(o[u], L)] = nd[u]
                gt = [nd[u] > vmaxs[u] for u in R]
                vmaxs = [jnp.where(gt[u], nd[u], vmaxs[u]) for u in R]
                jbests = [jnp.where(gt[u], jv, jbests[u]) for u in R]
                return tuple(vmaxs), tuple(jbests)

            neg1 = jnp.full((L,), -1.0, dtype=jnp.float32)
            zero = jnp.zeros((L,), dtype=jnp.int32)
            vmaxs, jbests = lax.fori_loop(
                0, CHUNKS // UNROLL, chunk_body,
                ((neg1,) * UNROLL, (zero,) * UNROLL))
            pairs = [(vmaxs[u], jbests[u] * (L * UNROLL) + (u * L) + lane)
                     for u in range(UNROLL)]
            while len(pairs) > 1:
                nxt = []
                for a in range(0, len(pairs), 2):
                    (va, ia), (vb, ib) = pairs[a], pairs[a + 1]
                    bwin = (vb > va) | ((vb == va) & (ib < ia))
                    nxt.append((jnp.where(bwin, vb, va),
                                jnp.where(bwin, ib, ia)))
                pairs = nxt
            vmax, vidx = pairs[0]
            m = jnp.max(vmax)
            cand = jnp.where(vmax == m, vidx, BIG_I32)
            return jnp.min(cand)

        lax.fori_loop(0, S, fps_iter, jnp.int32(0))

        def ps_body(j, _):
            base = j * L
            iv = idxb[pl.ds(base, L)]
            psx[pl.ds(base, L)] = plsc.load_gather(px, [iv])
            psy[pl.ds(base, L)] = plsc.load_gather(py, [iv])
            psz[pl.ds(base, L)] = plsc.load_gather(pz, [iv])
            return 0

        lax.fori_loop(0, S // L, ps_body, 0)
        pltpu.sync_copy(psx, p3_out.at[pl.ds((b * 3 + 0) * S, S)])
        pltpu.sync_copy(psy, p3_out.at[pl.ds((b * 3 + 1) * S, S)])
        pltpu.sync_copy(psz, p3_out.at[pl.ds((b * 3 + 2) * S, S)])

        def chan_body(ci, _):
            off = (b * C + ci) * N

            def bld(k, _):
                q = k // 8
                r = k - q * 8
                xidx[q, pl.ds(r * L, L)] = idxb[pl.ds(k * L, L)] + off
                return 0

            lax.fori_loop(0, S // L, bld, 0)

            _xgather(xflat_hbm, xidx, xg, sem)
            pltpu.sync_copy(xg, xs_out.at[pl.ds((b * C + ci) * S, S)])
            return 0

        lax.fori_loop(0, C, chan_body, 0)


@jax.jit
def kernel(p, x):
    p3 = jnp.transpose(p, (0, 2, 1)).reshape(B * 3 * N)
    xflat = x.reshape(B * C * N)

    mesh = plsc.VectorSubcoreMesh(core_axis_name="c", subcore_axis_name="s",
                                  num_cores=2, num_subcores=16)
    fps = pl.kernel(
        _fps_body,
        out_type=(
            jax.ShapeDtypeStruct((B * 3 * S,), jnp.float32),
            jax.ShapeDtypeStruct((B * C * S,), jnp.float32),
        ),
        mesh=mesh,
        compiler_params=pltpu.CompilerParams(needs_layout_passes=False),
        scratch_types=[
            pltpu.VMEM((N,), jnp.float32),
            pltpu.VMEM((N,), jnp.float32),
            pltpu.VMEM((N,), jnp.float32),
            pltpu.VMEM((N,), jnp.float32),
            pltpu.VMEM((S,), jnp.int32),
            pltpu.VMEM((S,), jnp.float32),
            pltpu.VMEM((S,), jnp.float32),
            pltpu.VMEM((S,), jnp.float32),
            pltpu.VMEM((8, 128), jnp.int32),
            pltpu.VMEM((S,), jnp.float32),
            pltpu.SemaphoreType.DMA,
        ],
    )
    p3_s, x_s = fps(p3, xflat)
    p_s = jnp.transpose(p3_s.reshape(B, 3, S), (0, 2, 1))
    return (p_s, x_s.reshape(B, C, S))


# speedup vs baseline: 5.4745x; 1.8173x over previous
import functools

import jax
import jax.numpy as jnp
from jax import lax
from jax.experimental import pallas as pl
from jax.experimental.pallas import tpu as pltpu
from jax.experimental.pallas import tpu_sc as plsc

B = 16
N = 16384
C = 64
S = 1024
L = 16
N2 = N // 2
CH2 = N2 // L
UNROLL = 8
S2 = S // 2
C2 = C // 2

BIG_I32 = 2**30


def _xgather(xflat_hbm, xidx, xg, sem):
    cps = [pltpu.async_copy(xflat_hbm.at[xidx.at[q]],
                            xg.at[pl.ds(q * 128, 128)], sem)
           for q in range(8)]
    for cp in cps:
        cp.wait()


def _fps_body(p3_hbm, xflat_hbm, p3_out, xs_out,
              px, py, pz, dist, idxb, psx, psy, psz, xidx, xg,
              stage, tmp, shm, sem):
    c = lax.axis_index("c")
    s = lax.axis_index("s")
    b = c * 8 + s // 2
    h = s % 2
    hbase = h * N2
    partner = s ^ 1

    pltpu.sync_copy(p3_hbm.at[pl.ds((b * 3 + 0) * N, N)], px)
    pltpu.sync_copy(p3_hbm.at[pl.ds((b * 3 + 1) * N, N)], py)
    pltpu.sync_copy(p3_hbm.at[pl.ds((b * 3 + 2) * N, N)], pz)

    inf16 = jnp.full((L,), jnp.inf, dtype=jnp.float32)

    def init_body(j, _):
        base = j * (L * UNROLL)
        for u in range(UNROLL):
            dist[pl.ds(base + u * L, L)] = inf16
        return 0

    lax.fori_loop(0, CH2 // UNROLL, init_body, 0)

    lane = lax.iota(jnp.int32, L)
    lane0 = lane == 0

    def fps_iter(si, fv):
        plsc.store_scatter(idxb, [jnp.full((L,), si, dtype=jnp.int32)],
                           fv, mask=lane0)
        cx = plsc.load_gather(px, [fv])
        cy = plsc.load_gather(py, [fv])
        cz = plsc.load_gather(pz, [fv])

        def chunk_body(j, carry):
            vmaxs, jbests = carry
            base = j * (L * UNROLL)
            jv = jnp.full((L,), j, dtype=jnp.int32)
            vmaxs = list(vmaxs)
            jbests = list(jbests)
            R = range(UNROLL)
            o = [base + u * L for u in R]
            g = [hbase + o[u] for u in R]
            xs = [px[pl.ds(g[u], L)] for u in R]
            ys = [py[pl.ds(g[u], L)] for u in R]
            zs = [pz[pl.ds(g[u], L)] for u in R]
            dv = [dist[pl.ds(o[u], L)] for u in R]
            dx = [xs[u] - cx for u in R]
            dy = [ys[u] - cy for u in R]
            dz = [zs[u] - cz for u in R]
            d = [dx[u] * dx[u] for u in R]
            d = [d[u] + dy[u] * dy[u] for u in R]
            d = [d[u] + dz[u] * dz[u] for u in R]
            nd = [jnp.minimum(dv[u], d[u]) for u in R]
            for u in R:
                dist[pl.ds(o[u], L)] = nd[u]
            gt = [nd[u] > vmaxs[u] for u in R]
            vmaxs = [jnp.where(gt[u], nd[u], vmaxs[u]) for u in R]
            jbests = [jnp.where(gt[u], jv, jbests[u]) for u in R]
            return tuple(vmaxs), tuple(jbests)

        neg1 = jnp.full((L,), -1.0, dtype=jnp.float32)
        zero = jnp.zeros((L,), dtype=jnp.int32)
        vmaxs, jbests = lax.fori_loop(
            0, CH2 // UNROLL, chunk_body,
            ((neg1,) * UNROLL, (zero,) * UNROLL))
        pairs = [(vmaxs[u],
                  jbests[u] * (L * UNROLL) + (hbase + u * L) + lane)
                 for u in range(UNROLL)]
        while len(pairs) > 1:
            nxt = []
            for a in range(0, len(pairs), 2):
                (va, ia), (vb, ib) = pairs[a], pairs[a + 1]
                bwin = (vb > va) | ((vb == va) & (ib < ia))
                nxt.append((jnp.where(bwin, vb, va),
                            jnp.where(bwin, ib, ia)))
            pairs = nxt
        vmax, vidx = pairs[0]
        m = jnp.max(vmax)
        cand = jnp.where(vmax == m, vidx, BIG_I32)
        il = jnp.min(cand)

        ml_v = jnp.full((L,), m, dtype=jnp.float32)
        il_v = jnp.full((L,), il, dtype=jnp.int32)
        par = si % 2
        stage[pl.ds(0, L)] = ml_v
        stage[pl.ds(L, L)] = plsc.bitcast(il_v, jnp.float32)
        pltpu.sync_copy(stage, shm.at[par, s])
        plsc.subcore_barrier()
        pltpu.sync_copy(shm.at[par, partner], tmp)
        mo_v = tmp[pl.ds(0, L)]
        io_v = plsc.bitcast(tmp[pl.ds(L, L)], jnp.int32)
        take_o = (mo_v > ml_v) | ((mo_v == ml_v) & (io_v < il_v))
        return jnp.where(take_o, io_v, il_v)

    lax.fori_loop(0, S, fps_iter, jnp.zeros((L,), dtype=jnp.int32))

    def ps_body(j, _):
        base = j * L
        iv = idxb[pl.ds(h * S2 + base, L)]
        psx[pl.ds(base, L)] = plsc.load_gather(px, [iv])
        psy[pl.ds(base, L)] = plsc.load_gather(py, [iv])
        psz[pl.ds(base, L)] = plsc.load_gather(pz, [iv])
        return 0

    lax.fori_loop(0, S2 // L, ps_body, 0)
    pltpu.sync_copy(psx, p3_out.at[pl.ds((b * 3 + 0) * S + h * S2, S2)])
    pltpu.sync_copy(psy, p3_out.at[pl.ds((b * 3 + 1) * S + h * S2, S2)])
    pltpu.sync_copy(psz, p3_out.at[pl.ds((b * 3 + 2) * S + h * S2, S2)])

    def chan_body(cl, _):
        ci = h * C2 + cl
        off = (b * C + ci) * N

        def bld(k, _):
            q = k // 8
            r = k - q * 8
            xidx[q, pl.ds(r * L, L)] = idxb[pl.ds(k * L, L)] + off
            return 0

        lax.fori_loop(0, S // L, bld, 0)

        _xgather(xflat_hbm, xidx, xg, sem)
        pltpu.sync_copy(xg, xs_out.at[pl.ds((b * C + ci) * S, S)])
        return 0

    lax.fori_loop(0, C2, chan_body, 0)


@jax.jit
def kernel(p, x):
    p3 = jnp.transpose(p, (0, 2, 1)).reshape(B * 3 * N)
    xflat = x.reshape(B * C * N)

    mesh = plsc.VectorSubcoreMesh(core_axis_name="c", subcore_axis_name="s",
                                  num_cores=2, num_subcores=16)
    fps = pl.kernel(
        _fps_body,
        out_type=(
            jax.ShapeDtypeStruct((B * 3 * S,), jnp.float32),
            jax.ShapeDtypeStruct((B * C * S,), jnp.float32),
        ),
        mesh=mesh,
        compiler_params=pltpu.CompilerParams(needs_layout_passes=False),
        scratch_types=[
            pltpu.VMEM((N,), jnp.float32),
            pltpu.VMEM((N,), jnp.float32),
            pltpu.VMEM((N,), jnp.float32),
            pltpu.VMEM((N2,), jnp.float32),
            pltpu.VMEM((S,), jnp.int32),
            pltpu.VMEM((S2,), jnp.float32),
            pltpu.VMEM((S2,), jnp.float32),
            pltpu.VMEM((S2,), jnp.float32),
            pltpu.VMEM((8, 128), jnp.int32),
            pltpu.VMEM((S,), jnp.float32),
            pltpu.VMEM((2 * L,), jnp.float32),
            pltpu.VMEM((2 * L,), jnp.float32),
            pltpu.VMEM_SHARED((2, 16, 2 * L), jnp.float32),
            pltpu.SemaphoreType.DMA,
        ],
    )
    p3_s, x_s = fps(p3, xflat)
    p_s = jnp.transpose(p3_s.reshape(B, 3, S), (0, 2, 1))
    return (p_s, x_s.reshape(B, C, S))
